# baseline (speedup 1.0000x reference)
import functools

import jax
import jax.numpy as jnp
from jax import lax
from jax.experimental import pallas as pl
from jax.experimental.pallas import tpu as pltpu
from jax.experimental.pallas import tpu_sc as plsc

N = 10000
N_PAD = 10240
E = 320000
E_PAD = 327680
EPT = E_PAD // 32
IN_C = 128
HID = 256
QUART = N_PAD // 4
SUB = 2944
CHUNK = 128
SUB_CHUNKS = 23
NPAIR = 12
NSUB = 16
SENT = 1 << 30
BN_SCALE = float(1.0 / (1.0 + 1e-5) ** 0.5)
ROWS_BLK = 512
GRID = N_PAD // ROWS_BLK


def _build_partition():
    mesh = plsc.VectorSubcoreMesh(core_axis_name="c", subcore_axis_name="s")

    @functools.partial(
        pl.kernel,
        out_type=(
            jax.ShapeDtypeStruct((4, 32, SUB), jnp.int32),
            jax.ShapeDtypeStruct((4, 32, SUB), jnp.int32),
            jax.ShapeDtypeStruct((4, 32, SUB), jnp.int32),
            jax.ShapeDtypeStruct((4, 32, SUB), jnp.int32),
        ),
        mesh=mesh,
        scratch_types=[
            pltpu.VMEM((EPT,), jnp.int32),
            pltpu.VMEM((EPT,), jnp.int32),
            [[pltpu.VMEM((SUB + 16,), jnp.int32) for _ in range(4)]
             for _ in range(4)],
        ],
        compiler_params=pltpu.CompilerParams(needs_layout_passes=False),
    )
    def part_kernel(srcE, dstE, psl, psh, pdl, pdh,
                    src_v, dst_v, bufs):
        c = lax.axis_index("c")
        s = lax.axis_index("s")
        w = c * NSUB + s
        pltpu.sync_copy(srcE.at[pl.ds(w * EPT, EPT)], src_v)
        pltpu.sync_copy(dstE.at[pl.ds(w * EPT, EPT)], dst_v)

        iota16 = lax.iota(jnp.int32, 16)
        fills = (jnp.full((16,), 2 * N, jnp.int32),
                 jnp.full((16,), 2 * N + 1, jnp.int32),
                 jnp.zeros((16,), jnp.int32),
                 jnp.full((16,), 1, jnp.int32))

        @pl.loop(0, SUB // 16 + 1)
        def _fill(i):
            for q in range(4):
                for a in range(4):
                    bufs[q][a][pl.ds(i * 16, 16)] = fills[a]

        @pl.loop(0, EPT // 16,
                 init_carry=tuple(jnp.int32(0) for _ in range(4)))
        def offs(i, carry):
            sv = src_v[pl.ds(i * 16, 16)]
            dv = dst_v[pl.ds(i * 16, 16)]
            k = jnp.minimum(dv // QUART, 4).astype(jnp.int32)
            _, svs = plsc.sort_key_val(k, sv)
            _, dvs = plsc.sort_key_val(k, dv)
            cq = [jnp.max(plsc.all_reduce_population_count(k == q))
                  for q in range(4)]
            start = jnp.int32(0)
            out = []
            for q in range(4):
                rot = jnp.remainder(iota16 + start, 16)
                s_r = svs.at[rot].get(mode="promise_in_bounds")
                d_r = dvs.at[rot].get(mode="promise_in_bounds") - q * QUART
                off = carry[q]
                bufs[q][0][pl.ds(off, 16)] = 2 * s_r
                bufs[q][1][pl.ds(off, 16)] = 2 * s_r + 1
                bufs[q][2][pl.ds(off, 16)] = 2 * d_r
                bufs[q][3][pl.ds(off, 16)] = 2 * d_r + 1
                out.append(off + cq[q])
                start = start + cq[q]
            return tuple(out)

        offq = offs
        outs = (psl, psh, pdl, pdh)
        for q in range(4):
            for a in range(4):
                bufs[q][a][pl.ds(offq[q], 16)] = fills[a]
                pltpu.sync_copy(bufs[q][a].at[pl.ds(0, SUB)],
                                outs[a].at[q, w])

    return part_kernel


def _build_consumer():
    rpt = 2 * QUART // NSUB
    mesh = plsc.VectorSubcoreMesh(core_axis_name="c", subcore_axis_name="s")

    @functools.partial(
        pl.kernel,
        out_type=jax.ShapeDtypeStruct((2, 2 * QUART, 128), jnp.float32),
        mesh=mesh,
        scratch_types=[
            [pltpu.VMEM((SUB_CHUNKS + 3, CHUNK), jnp.int32)
             for _ in range(4)],
            [pltpu.VMEM((CHUNK, 128), jnp.float32) for _ in range(4)],
            pltpu.VMEM((8, 128), jnp.float32),
            pltpu.VMEM_SHARED((2 * QUART, 128), jnp.float32),
            [pltpu.SemaphoreType.DMA for _ in range(4)],
            pltpu.SemaphoreType.DMA,
        ],
    )
    def consumer(table, psl, psh, pdl, pdh, out,
                 idx_v, gb, zb, acc_sh, gsem, ssem):
        c = lax.axis_index("c")
        s = lax.axis_index("s")

        zeros16 = jnp.zeros((16,), jnp.float32)
        for i in range(8):
            for k in range(8):
                zb[i, pl.ds(k * 16, 16)] = zeros16

        @pl.loop(0, rpt // 8)
        def _zero(j):
            pltpu.sync_copy(zb, acc_sh.at[pl.ds(s * rpt + j * 8, 8)])

        plsc.subcore_barrier()

        fills = (jnp.full((16,), 2 * N, jnp.int32),
                 jnp.full((16,), 2 * N + 1, jnp.int32),
                 jnp.zeros((16,), jnp.int32),
                 jnp.full((16,), 1, jnp.int32))
        for r in range(3):
            for k in range(CHUNK // 16):
                for a in range(4):
                    idx_v[a][SUB_CHUNKS + r, pl.ds(k * 16, 16)] = fills[a]

        def gather(j, b, p):
            pltpu.async_copy(table.at[idx_v[p].at[j]],
                             gb[2 * b + p], gsem[2 * b + p])

        def gwait(j, b, p):
            pltpu.make_async_copy(table.at[idx_v[p].at[j]],
                                  gb[2 * b + p], gsem[2 * b + p]).wait()

        for sub_i in range(2):
            sub = 2 * s + sub_i
            for a, src in ((0, psl), (1, psh), (2, pdl), (3, pdh)):
                pltpu.sync_copy(src.at[c, sub],
                                idx_v[a].at[pl.ds(0, SUB_CHUNKS)])
            hi_lim = (jnp.full((16,), 2 * N_PAD - 1, jnp.int32),
                      jnp.full((16,), 2 * QUART - 1, jnp.int32))

            @pl.loop(0, SUB_CHUNKS)
            def _clamp(r):
                for a in range(4):
                    for k in range(CHUNK // 16):
                        v = idx_v[a][r, pl.ds(k * 16, 16)]
                        v = jnp.minimum(jnp.maximum(v, 0), hi_lim[a // 2])
                        idx_v[a][r, pl.ds(k * 16, 16)] = v
            for b in (0, 1):
                gather(b, b, 0)
                gather(b, b, 1)

            @pl.loop(0, NPAIR)
            def _pair(q):
                for b in (0, 1):
                    j = 2 * q + b
                    for p in (0, 1):
                        gwait(j, b, p)
                        pltpu.async_copy(
                            gb[2 * b + p],
                            acc_sh.at[idx_v[2 + p].at[j]],
                            ssem, add=True).wait()
                    gather(j + 2, b, 0)
                    gather(j + 2, b, 1)

            for b in (0, 1):
                gwait(2 * NPAIR + b, b, 0)
                gwait(2 * NPAIR + b, b, 1)

        plsc.subcore_barrier()
        pltpu.sync_copy(acc_sh.at[pl.ds(s * rpt, rpt)],
                        out.at[c, pl.ds(s * rpt, rpt)])

    return consumer


_build_partition = functools.lru_cache(maxsize=None)(_build_partition)
_build_consumer = functools.lru_cache(maxsize=None)(_build_consumer)


def _ln_relu(x, g, b):
    mu = jnp.mean(x, axis=-1, keepdims=True)
    xc = x - mu
    var = jnp.mean(xc * xc, axis=-1, keepdims=True)
    return jnp.maximum(xc * lax.rsqrt(var + 1e-5) * g + b, 0.0)


def _row_mask(h):
    i = pl.program_id(0)
    rows = i * ROWS_BLK + lax.broadcasted_iota(jnp.int32, (ROWS_BLK, 1), 0)
    return jnp.where(rows < N, h, 0.0)


def _tc_first_body(e1_ref, f_ref, agg_ref, w1_ref, b1_ref, w2_ref, b2_ref,
                   g_ref, bb_ref, x_ref, ht_ref):
    u = e1_ref[...] * f_ref[...] + agg_ref[...]
    t = jnp.dot(u, w1_ref[...], preferred_element_type=jnp.float32) + b1_ref[...]
    t = jnp.maximum(t * BN_SCALE, 0.0)
    x = jnp.dot(t, w2_ref[...], preferred_element_type=jnp.float32) + b2_ref[...]
    x_ref[...] = x
    ht_ref[...] = _row_mask(_ln_relu(x, g_ref[...], bb_ref[...]))


def _tc_mid_body(e1_ref, xp_ref, agg_ref, w1_ref, b1_ref, w2_ref, b2_ref,
                 gp_ref, bp_ref, g_ref, bb_ref, x_ref, ht_ref):
    xp = xp_ref[...]
    h = _ln_relu(xp, gp_ref[...], bp_ref[...])
    u = e1_ref[...] * h + agg_ref[...]
    t = jnp.dot(u, w1_ref[...], preferred_element_type=jnp.float32) + b1_ref[...]
    t = jnp.maximum(t * BN_SCALE, 0.0)
    x = xp + jnp.dot(t, w2_ref[...], preferred_element_type=jnp.float32) + b2_ref[...]
    x_ref[...] = x
    ht_ref[...] = _row_mask(_ln_relu(x, g_ref[...], bb_ref[...]))


def _row_spec(d):
    return pl.BlockSpec((ROWS_BLK, d), lambda i: (i, 0))


def _full_spec(r, d):
    return pl.BlockSpec((r, d), lambda i: (0, 0))


_OUT_SHAPES = (
    jax.ShapeDtypeStruct((N_PAD, HID), jnp.float32),
    jax.ShapeDtypeStruct((N_PAD, HID), jnp.float32),
)
_OUT_SPECS = (_row_spec(HID), _row_spec(HID))

_tc_first = pl.pallas_call(
    _tc_first_body,
    grid=(GRID,),
    in_specs=[
        _full_spec(1, 1),
        _row_spec(HID), _row_spec(HID),
        _full_spec(HID, HID), _full_spec(1, HID),
        _full_spec(HID, HID), _full_spec(1, HID),
        _full_spec(1, HID), _full_spec(1, HID),
    ],
    out_specs=_OUT_SPECS,
    out_shape=_OUT_SHAPES,
)

_tc_mid = pl.pallas_call(
    _tc_mid_body,
    grid=(GRID,),
    in_specs=[
        _full_spec(1, 1),
        _row_spec(HID), _row_spec(HID),
        _full_spec(HID, HID), _full_spec(1, HID),
        _full_spec(HID, HID), _full_spec(1, HID),
        _full_spec(1, HID), _full_spec(1, HID),
        _full_spec(1, HID), _full_spec(1, HID),
    ],
    out_specs=_OUT_SPECS,
    out_shape=_OUT_SHAPES,
)


def kernel(feature_vector, adj_index, edge_vector, params):
    del edge_vector
    src = adj_index[0]
    dst = adj_index[1]
    pad = E_PAD - E
    src_e = jnp.concatenate([src, jnp.full((pad,), N, jnp.int32)])
    dst_e = jnp.concatenate([dst, jnp.full((pad,), SENT, jnp.int32)])

    parts = _build_partition()(src_e, dst_e)
    psl, psh, pdl, pdh = (p.reshape(4, 32, SUB_CHUNKS, CHUNK)
                          for p in parts)

    f_pad = jnp.zeros((N_PAD, HID), jnp.float32
                      ).at[:N, :IN_C].set(feature_vector)
    layers = params["layers"]

    def wb(p):
        return (p["W1"], p["b1"].reshape(1, HID), p["W2"],
                p["b2"].reshape(1, HID))

    def ln(p):
        return p["ln_g"].reshape(1, HID), p["ln_b"].reshape(1, HID)

    def e1(p):
        return (1.0 + p["eps"]).reshape(1, 1)

    consume = _build_consumer()

    def agg_of(tbl):
        tbl2 = tbl.reshape(2 * N_PAD, 128)
        lo = consume(tbl2, psl[0:2], psh[0:2], pdl[0:2], pdh[0:2])
        hi = consume(tbl2, psl[2:4], psh[2:4], pdl[2:4], pdh[2:4])
        return jnp.concatenate([lo, hi], axis=0).reshape(N_PAD, HID)

    p0 = layers[0]
    w1, b1, w2, b2 = wb(p0)
    w1 = jnp.zeros((HID, HID), jnp.float32).at[:IN_C].set(w1)
    g, b = ln(layers[1])
    agg = agg_of(f_pad)
    x, ht = _tc_first(e1(p0), f_pad, agg, w1, b1, w2, b2, g, b)

    for l in (1, 2, 3):
        pl_ = layers[l]
        agg = agg_of(ht)
        w1, b1, w2, b2 = wb(pl_)
        gp, bp = ln(pl_)
        g, b = ln(layers[l + 1] if l < 3 else layers[0])
        x, ht = _tc_mid(e1(pl_), x, agg, w1, b1, w2, b2, gp, bp, g, b)

    return ht[:N]


# speedup vs baseline: 6.4646x; 6.4646x over previous
import functools

import jax
import jax.numpy as jnp
from jax import lax
from jax.experimental import pallas as pl
from jax.experimental.pallas import tpu as pltpu
from jax.experimental.pallas import tpu_sc as plsc

N = 10000
N_PAD = 10240
E = 320000
E_PAD = 327680
IN_C = 128
HID = 256
HALF = 128
CHUNK = 64
NUM_CHUNKS = E_PAD // CHUNK
NBUF = 4
NSUB = 16
ROWS_PER_TILE = N_PAD // NSUB
BN_SCALE = float(1.0 / (1.0 + 1e-5) ** 0.5)
ROWS_BLK = 512
GRID = N_PAD // ROWS_BLK


def _make_agg(edge_split: bool):
    cpt = NUM_CHUNKS // 32 if edge_split else NUM_CHUNKS // 16
    stage = 40
    n_stages = cpt // stage
    mesh = plsc.VectorSubcoreMesh(core_axis_name="c", subcore_axis_name="s")

    @functools.partial(
        pl.kernel,
        out_type=jax.ShapeDtypeStruct((2, N_PAD, HALF), jnp.float32),
        mesh=mesh,
        scratch_types=[
            pltpu.VMEM((stage, CHUNK), jnp.int32),
            pltpu.VMEM((stage, CHUNK), jnp.int32),
            [pltpu.VMEM((CHUNK, HALF), jnp.float32) for _ in range(NBUF)],
            pltpu.VMEM((16, HALF), jnp.float32),
            pltpu.VMEM_SHARED((N_PAD, HALF), jnp.float32),
            [pltpu.SemaphoreType.DMA for _ in range(NBUF)],
            [pltpu.SemaphoreType.DMA for _ in range(NBUF)],
        ],
    )
    def agg_kernel(t0_hbm, t1_hbm, src_hbm, dst_hbm, out_hbm,
                   src_v, dst_v, rows_v, zb_v, acc_sh, gsem, ssem):
        c = lax.axis_index("c")
        s = lax.axis_index("s")
        if edge_split:
            base = c * (NUM_CHUNKS // 2) + s * cpt
        else:
            base = s * cpt

        zeros16 = jnp.zeros((16,), jnp.float32)
        for i in range(16):
            for k in range(HALF // 16):
                zb_v[i, pl.ds(k * 16, 16)] = zeros16

        @pl.loop(0, ROWS_PER_TILE // 16)
        def _zero(j):
            pltpu.sync_copy(zb_v, acc_sh.at[pl.ds(s * ROWS_PER_TILE + j * 16, 16)])

        plsc.subcore_barrier()

        def run_edges(tbl):
            @pl.loop(0, n_stages)
            def _stage(g):
                pltpu.sync_copy(src_hbm.at[pl.ds(base + g * stage, stage)], src_v)
                pltpu.sync_copy(dst_hbm.at[pl.ds(base + g * stage, stage)], dst_v)

                @pl.loop(0, stage // NBUF)
                def _group(q):
                    j = q * NBUF
                    gd = [pltpu.async_copy(tbl.at[src_v.at[j + b]],
                                           rows_v[b], gsem[b])
                          for b in range(NBUF)]
                    sd = []
                    for b in range(NBUF):
                        gd[b].wait()
                        sd.append(pltpu.async_copy(
                            rows_v[b], acc_sh.at[dst_v.at[j + b]],
                            ssem[b], add=True))
                    for b in range(NBUF):
                        sd[b].wait()

        if edge_split:
            run_edges(t0_hbm)
        else:
            @pl.when(c == 0)
            def _():
                run_edges(t0_hbm)

            @pl.when(c == 1)
            def _():
                run_edges(t1_hbm)

        plsc.subcore_barrier()
        pltpu.sync_copy(
            acc_sh.at[pl.ds(s * ROWS_PER_TILE, ROWS_PER_TILE)],
            out_hbm.at[c, pl.ds(s * ROWS_PER_TILE, ROWS_PER_TILE)])

    return agg_kernel


_make_agg = functools.lru_cache(maxsize=None)(_make_agg)


def _agg_edge_split(t0, t1, src_p, dst_p):
    return _make_agg(True)(t0, t1, src_p, dst_p)


def _agg_chan_split(t0, t1, src_p, dst_p):
    return _make_agg(False)(t0, t1, src_p, dst_p)


def _ln_relu(x, g, b):
    mu = jnp.mean(x, axis=-1, keepdims=True)
    xc = x - mu
    var = jnp.mean(xc * xc, axis=-1, keepdims=True)
    return jnp.maximum(xc * lax.rsqrt(var + 1e-5) * g + b, 0.0)


def _tc_layer0_body(e1_ref, f_ref, p0_ref, p1_ref, w1_ref, b1_ref,
                    w2_ref, b2_ref, g_ref, bb_ref,
                    x_ref, hlo_ref, hhi_ref):
    u = e1_ref[...] * f_ref[...] + (p0_ref[...] + p1_ref[...])
    t = jnp.dot(u, w1_ref[...], preferred_element_type=jnp.float32) + b1_ref[...]
    t = jnp.maximum(t * BN_SCALE, 0.0)
    x = jnp.dot(t, w2_ref[...], preferred_element_type=jnp.float32) + b2_ref[...]
    x_ref[...] = x
    h = _ln_relu(x, g_ref[...], bb_ref[...])
    hlo_ref[...] = h[:, :HALF]
    hhi_ref[...] = h[:, HALF:]


def _tc_layer_body(e1_ref, hlo_ref, hhi_ref, alo_ref, ahi_ref, xp_ref,
                   w1_ref, b1_ref, w2_ref, b2_ref, g_ref, bb_ref,
                   x_ref, olo_ref, ohi_ref):
    e1 = e1_ref[...]
    ulo = e1 * hlo_ref[...] + alo_ref[...]
    uhi = e1 * hhi_ref[...] + ahi_ref[...]
    t = (jnp.dot(ulo, w1_ref[:HALF, :], preferred_element_type=jnp.float32)
         + jnp.dot(uhi, w1_ref[HALF:, :], preferred_element_type=jnp.float32)
         + b1_ref[...])
    t = jnp.maximum(t * BN_SCALE, 0.0)
    y = jnp.dot(t, w2_ref[...], preferred_element_type=jnp.float32) + b2_ref[...]
    x = xp_ref[...] + y
    x_ref[...] = x
    h = _ln_relu(x, g_ref[...], bb_ref[...])
    olo_ref[...] = h[:, :HALF]
    ohi_ref[...] = h[:, HALF:]


def _row_spec(d):
    return pl.BlockSpec((ROWS_BLK, d), lambda i: (i, 0))


def _full_spec(r, d):
    return pl.BlockSpec((r, d), lambda i: (0, 0))


_OUT_SHAPES = (
    jax.ShapeDtypeStruct((N_PAD, HID), jnp.float32),
    jax.ShapeDtypeStruct((N_PAD, HALF), jnp.float32),
    jax.ShapeDtypeStruct((N_PAD, HALF), jnp.float32),
)
_OUT_SPECS = (_row_spec(HID), _row_spec(HALF), _row_spec(HALF))

_tc_layer0 = pl.pallas_call(
    _tc_layer0_body,
    grid=(GRID,),
    in_specs=[
        _full_spec(1, 1),
        _row_spec(IN_C), _row_spec(HALF), _row_spec(HALF),
        _full_spec(IN_C, HID), _full_spec(1, HID),
        _full_spec(HID, HID), _full_spec(1, HID),
        _full_spec(1, HID), _full_spec(1, HID),
    ],
    out_specs=_OUT_SPECS,
    out_shape=_OUT_SHAPES,
)

_tc_layer = pl.pallas_call(
    _tc_layer_body,
    grid=(GRID,),
    in_specs=[
        _full_spec(1, 1),
        _row_spec(HALF), _row_spec(HALF), _row_spec(HALF), _row_spec(HALF),
        _row_spec(HID),
        _full_spec(HID, HID), _full_spec(1, HID),
        _full_spec(HID, HID), _full_spec(1, HID),
        _full_spec(1, HID), _full_spec(1, HID),
    ],
    out_specs=_OUT_SPECS,
    out_shape=_OUT_SHAPES,
)


def kernel(feature_vector, adj_index, edge_vector, params):
    del edge_vector
    src = adj_index[0]
    dst = adj_index[1]
    f = jnp.zeros((N_PAD, IN_C), jnp.float32).at[:N].set(feature_vector)
    pad = E_PAD - E
    src_p = jnp.concatenate([src, jnp.zeros((pad,), jnp.int32)]
                            ).reshape(NUM_CHUNKS, CHUNK)
    dst_p = jnp.concatenate([dst, jnp.full((pad,), N, jnp.int32)]
                            ).reshape(NUM_CHUNKS, CHUNK)
    layers = params["layers"]

    def wb(p):
        return (p["W1"], p["b1"].reshape(1, HID), p["W2"],
                p["b2"].reshape(1, HID))

    def ln(p):
        return p["ln_g"].reshape(1, HID), p["ln_b"].reshape(1, HID)

    def e1(p):
        return (1.0 + p["eps"]).reshape(1, 1)

    p0 = layers[0]
    parts = _agg_edge_split(f, f, src_p, dst_p)
    w1, b1, w2, b2 = wb(p0)
    g, b = ln(layers[1])
    x, hlo, hhi = _tc_layer0(e1(p0), f, parts[0], parts[1],
                             w1, b1, w2, b2, g, b)

    for l in (1, 2, 3):
        pl_ = layers[l]
        agg = _agg_chan_split(hlo, hhi, src_p, dst_p)
        w1, b1, w2, b2 = wb(pl_)
        g, b = ln(layers[l + 1] if l < 3 else layers[0])
        x, hlo, hhi = _tc_layer(e1(pl_), hlo, hhi, agg[0], agg[1], x,
                                w1, b1, w2, b2, g, b)

    return jnp.concatenate([hlo, hhi], axis=1)[:N]
